# baseline (speedup 1.0000x reference)
import functools

import jax
import jax.numpy as jnp
from jax import lax
from jax.experimental import pallas as pl
from jax.experimental.pallas import tpu as pltpu
from jax.experimental.pallas import tpu_sc as plsc

_ALPHA = 0.25
_A_TOTAL = 49104
_A_PAD = 49152
_C = 80
_G = 20
_BA = 4096
_TC_LIM = 40912
_TC_NB = 10
_SC_N = _A_TOTAL - _TC_LIM
_SC_PER_W = _SC_N // 16
_SC_NCH = 2
_SC_CHA = _SC_PER_W // _SC_NCH
_SC_CH = _SC_CHA * _C

_L2C = (-2.786812953867443, 5.046876044975941, -3.49249427987935,
        1.5939013634991297, -0.4048671744191854, 0.043428907822139526)
_LN2 = 0.6931471805599453


def _assign_body(ann_ref, anc_ref, reg_ref, acc_ref):
    j = pl.program_id(0)
    sa = anc_ref.shape[1]

    ax1 = anc_ref[0]
    ay1 = anc_ref[1]
    ax2 = anc_ref[2]
    ay2 = anc_ref[3]
    area_a = (ax2 - ax1) * (ay2 - ay1)

    best = jnp.full((sa, 128), -2.0, jnp.float32)
    gx1 = jnp.zeros((sa, 128), jnp.float32)
    gy1 = gx1
    gx2 = gx1
    gy2 = gx1
    for g in range(_G):
        bx1 = ann_ref[j, g, 0]
        by1 = ann_ref[j, g, 1]
        bx2 = ann_ref[j, g, 2]
        by2 = ann_ref[j, g, 3]
        bcl = ann_ref[j, g, 4]
        iw = jnp.clip(jnp.minimum(ax2, bx2) - jnp.maximum(ax1, bx1), 0.0, None)
        ih = jnp.clip(jnp.minimum(ay2, by2) - jnp.maximum(ay1, by1), 0.0, None)
        inter = iw * ih
        ua = jnp.clip(area_a + (bx2 - bx1) * (by2 - by1) - inter, 1e-8, None)
        iou = jnp.where(bcl != -1.0, inter / ua, -1.0)
        upd = iou > best
        best = jnp.where(upd, iou, best)
        gx1 = jnp.where(upd, bx1, gx1)
        gy1 = jnp.where(upd, by1, gy1)
        gx2 = jnp.where(upd, bx2, gx2)
        gy2 = jnp.where(upd, by2, gy2)

    lane = jax.lax.broadcasted_iota(jnp.int32, (sa, 128), 1)
    sub = jax.lax.broadcasted_iota(jnp.int32, (sa, 128), 0)
    rv = sub * 128 + lane < _A_TOTAL
    posv = jnp.logical_and(best >= 0.5, rv)

    aw = ax2 - ax1
    ah = ay2 - ay1
    gwr = gx2 - gx1
    ghr = gy2 - gy1
    tdx = (gx1 + 0.5 * gwr - ax1 - 0.5 * aw) / aw * 10.0
    tdy = (gy1 + 0.5 * ghr - ay1 - 0.5 * ah) / ah * 10.0
    tdw = jnp.log(jnp.clip(gwr, 1.0, None) / aw) * 5.0
    tdh = jnp.log(jnp.clip(ghr, 1.0, None) / ah) * 5.0

    def _sl1(d):
        ad = jnp.abs(d)
        return jnp.where(ad <= 1.0 / 9.0, 4.5 * ad * ad, ad - 0.5 / 9.0)

    rl = (_sl1(tdx - reg_ref[0, 0]) + _sl1(tdy - reg_ref[0, 1])
          + _sl1(tdw - reg_ref[0, 2]) + _sl1(tdh - reg_ref[0, 3]))
    reg_part = jnp.sum(jnp.where(posv, rl, 0.0))
    pos_part = jnp.sum(jnp.where(posv, 1.0, 0.0))
    c128 = jax.lax.broadcasted_iota(jnp.int32, (8, 128), 1)
    acc_ref[0] = (jnp.where(c128 == 1, reg_part, 0.0)
                  + jnp.where(c128 == 2, pos_part, 0.0))


def _cls_body(ann_ref, cls_ref, anc_ref, out_ref):
    i = pl.program_id(1)

    @pl.when(i == 0)
    def _init():
        out_ref[...] = jnp.zeros_like(out_ref)

    bx1 = ann_ref[0, 0:1, :]
    by1 = ann_ref[0, 1:2, :]
    bx2 = ann_ref[0, 2:3, :]
    by2 = ann_ref[0, 3:4, :]
    bcl = ann_ref[0, 4:5, :]

    ax1 = anc_ref[0, :, 0:1]
    ay1 = anc_ref[0, :, 1:2]
    ax2 = anc_ref[0, :, 2:3]
    ay2 = anc_ref[0, :, 3:4]

    iw = jnp.clip(jnp.minimum(ax2, bx2) - jnp.maximum(ax1, bx1), 0.0, None)
    ih = jnp.clip(jnp.minimum(ay2, by2) - jnp.maximum(ay1, by1), 0.0, None)
    inter = iw * ih
    area_b = (bx2 - bx1) * (by2 - by1)
    area_a = (ax2 - ax1) * (ay2 - ay1)
    ua = jnp.clip(area_a + area_b - inter, 1e-8, None)
    iou = jnp.where(bcl != -1.0, inter / ua, -1.0)

    iou_max = jnp.max(iou, axis=1, keepdims=True)
    g_iota = jax.lax.broadcasted_iota(jnp.int32, iou.shape, 1)
    first_arg = jnp.min(jnp.where(iou >= iou_max, g_iota, 2**30), axis=1,
                        keepdims=True)
    eq = (g_iota == first_arg).astype(jnp.float32)
    gcl = jnp.sum(eq * bcl, axis=1, keepdims=True)

    row = jax.lax.broadcasted_iota(jnp.int32, iou_max.shape, 0) + i * _BA
    bnd = row < _TC_LIM
    pc = jnp.logical_and(iou_max >= 0.5, bnd)
    kc = jnp.logical_and(jnp.logical_or(iou_max < 0.4, iou_max >= 0.5), bnd)
    ic = gcl.astype(jnp.int32)

    c = jnp.clip(cls_ref[0], 1e-4, 1.0 - 1e-4)
    neg = (1.0 - _ALPHA) * c * c * (-jnp.log(1.0 - c))
    c_iota = jax.lax.broadcasted_iota(jnp.int32, c.shape, 1)
    onehot = c_iota == ic
    s1 = jnp.sum(jnp.where(kc, neg, 0.0))
    s_negt = jnp.sum(jnp.where(jnp.logical_and(onehot, pc), neg, 0.0))
    ct = jnp.sum(jnp.where(onehot, c, 0.0), axis=1, keepdims=True)
    ct = jnp.clip(ct, 1e-4, 1.0 - 1e-4)
    post = _ALPHA * (1.0 - ct) * (1.0 - ct) * (-jnp.log(ct))
    s_post = jnp.sum(jnp.where(pc, post, 0.0))
    cls_part = s1 - s_negt + s_post

    c128 = jax.lax.broadcasted_iota(jnp.int32, (8, 128), 1)
    out_ref[0] += jnp.where(c128 == 0, cls_part, 0.0)


def _fast_log(y):
    bits = plsc.bitcast(y, jnp.int32)
    e = (jnp.right_shift(bits, 23) - 127).astype(jnp.float32)
    m = plsc.bitcast(
        jnp.bitwise_or(jnp.bitwise_and(bits, 0x007FFFFF), 0x3F800000),
        jnp.float32)
    p = jnp.float32(_L2C[5])
    for k in (4, 3, 2, 1, 0):
        p = p * m + jnp.float32(_L2C[k])
    return (e + p) * jnp.float32(_LN2)


def _sc_cls_body(cls_hbm, anc_hbm, ann_hbm, out_hbm,
                 buf0, buf1, ancbuf, abuf, accv, sem0, sem1):
    w = lax.axis_index("s") * 2 + lax.axis_index("c")
    b = w // 16
    wis = w % 16
    abase = _TC_LIM + wis * _SC_PER_W
    ebase = b * (_A_TOTAL * _C) + abase * _C

    pltpu.sync_copy(ann_hbm, abuf)
    pltpu.sync_copy(anc_hbm.at[pl.ds(abase * 4, _SC_PER_W * 4)], ancbuf)

    bufs = (buf0, buf1)
    sems = (sem0, sem1)
    pltpu.make_async_copy(cls_hbm.at[pl.ds(ebase, _SC_CH)], buf0, sem0).start()

    iv = lax.iota(jnp.int32, 16) * _C
    iv4 = lax.iota(jnp.int32, 16) * 4
    zero16 = jnp.zeros((16,), jnp.int32)
    carry0 = (jnp.zeros((16,), jnp.float32),) * 3

    def _mk_group(cur, coff):
        def _group(g, carry):
            sneg, snt, spost = carry
            la4 = (coff + g * 16) * 4
            ax1 = plsc.load_gather(ancbuf, [iv4 + la4])
            ay1 = plsc.load_gather(ancbuf, [iv4 + (la4 + 1)])
            ax2 = plsc.load_gather(ancbuf, [iv4 + (la4 + 2)])
            ay2 = plsc.load_gather(ancbuf, [iv4 + (la4 + 3)])
            area_a = (ax2 - ax1) * (ay2 - ay1)
            best = jnp.full((16,), -2.0, jnp.float32)
            gcl = jnp.zeros((16,), jnp.float32)
            for gt in range(_G):
                bann = b * (_G * 5) + gt * 5
                bx1 = plsc.load_gather(abuf, [zero16 + bann])
                by1 = plsc.load_gather(abuf, [zero16 + (bann + 1)])
                bx2 = plsc.load_gather(abuf, [zero16 + (bann + 2)])
                by2 = plsc.load_gather(abuf, [zero16 + (bann + 3)])
                bcl = plsc.load_gather(abuf, [zero16 + (bann + 4)])
                iw = jnp.maximum(jnp.minimum(ax2, bx2) - jnp.maximum(ax1, bx1),
                                 0.0)
                ih = jnp.maximum(jnp.minimum(ay2, by2) - jnp.maximum(ay1, by1),
                                 0.0)
                inter = iw * ih
                ua = jnp.maximum(
                    area_a + (bx2 - bx1) * (by2 - by1) - inter, 1e-8)
                iou = jnp.where(bcl != -1.0, inter / ua, -1.0)
                upd = iou > best
                best = jnp.where(upd, iou, best)
                gcl = jnp.where(upd, bcl, gcl)
            posb = best >= 0.5
            keepb = jnp.logical_or(best < 0.4, posb)
            eidx = iv + g * (16 * _C)
            for k in range(_C):
                ck = plsc.load_gather(cur, [eidx + k])
                cc = jnp.minimum(jnp.maximum(ck, 1e-4), 1.0 - 1e-4)
                negk = (_ALPHA - 1.0) * cc * cc * _fast_log(1.0 - cc)
                sneg = sneg + jnp.where(keepb, negk, 0.0)
            ct = plsc.load_gather(
                cur, [eidx + jnp.maximum(gcl, 0.0).astype(jnp.int32)])
            ct = jnp.minimum(jnp.maximum(ct, 1e-4), 1.0 - 1e-4)
            negt = (_ALPHA - 1.0) * ct * ct * _fast_log(1.0 - ct)
            snt = snt + jnp.where(posb, negt, 0.0)
            post = -_ALPHA * (1.0 - ct) * (1.0 - ct) * _fast_log(ct)
            spost = spost + jnp.where(posb, post, 0.0)
            return (sneg, snt, spost)
        return _group

    carry = carry0
    for c in range(_SC_NCH):
        if c + 1 < _SC_NCH:
            pltpu.make_async_copy(
                cls_hbm.at[pl.ds(ebase + (c + 1) * _SC_CH, _SC_CH)],
                bufs[(c + 1) % 2], sems[(c + 1) % 2]).start()
        pltpu.make_async_copy(cls_hbm.at[pl.ds(ebase + c * _SC_CH, _SC_CH)],
                              bufs[c % 2], sems[c % 2]).wait()
        carry = lax.fori_loop(0, _SC_CHA // 16,
                              _mk_group(bufs[c % 2], c * _SC_CHA), carry)

    sneg, snt, spost = carry
    accv[...] = sneg - snt + spost
    pltpu.sync_copy(accv, out_hbm.at[w])


@functools.partial(jax.jit, static_argnames=("interpret",))
def kernel(classifications, regressions, anchors, annotations, image,
           interpret=False):
    del image
    B, A, C = classifications.shape
    sa = _A_PAD // 128

    mesh = plsc.VectorSubcoreMesh(core_axis_name="c", subcore_axis_name="s")
    sc_cls = functools.partial(
        pl.kernel,
        out_type=jax.ShapeDtypeStruct((32, 16), jnp.float32),
        mesh=mesh,
        scratch_types=[
            pltpu.VMEM((_SC_CH,), jnp.float32),
            pltpu.VMEM((_SC_CH,), jnp.float32),
            pltpu.VMEM((_SC_PER_W * 4,), jnp.float32),
            pltpu.VMEM((B * _G * 5,), jnp.float32),
            pltpu.VMEM((16,), jnp.float32),
            pltpu.SemaphoreType.DMA,
            pltpu.SemaphoreType.DMA,
        ],
        compiler_params=pltpu.CompilerParams(needs_layout_passes=False),
        interpret=interpret,
    )(_sc_cls_body)
    sc_acc = sc_cls(jnp.reshape(classifications, (-1,)),
                    jnp.reshape(anchors[0], (-1,)),
                    jnp.reshape(annotations, (-1,)))

    anc_t = jnp.swapaxes(anchors[0], 0, 1)
    anc_t = jnp.reshape(jnp.pad(anc_t, ((0, 0), (0, _A_PAD - A))),
                        (4, sa, 128))
    reg_t = jnp.swapaxes(regressions, 1, 2)
    reg_t = jnp.reshape(jnp.pad(reg_t, ((0, 0), (0, 0), (0, _A_PAD - A))),
                        (B, 4, sa, 128))
    acc_a = pl.pallas_call(
        _assign_body,
        grid=(B,),
        in_specs=[
            pl.BlockSpec(memory_space=pltpu.SMEM),
            pl.BlockSpec((4, sa, 128), lambda j: (0, 0, 0)),
            pl.BlockSpec((1, 4, sa, 128), lambda j: (j, 0, 0, 0)),
        ],
        out_specs=pl.BlockSpec((1, 8, 128), lambda j: (j, 0, 0)),
        out_shape=jax.ShapeDtypeStruct((B, 8, 128), jnp.float32),
        compiler_params=pltpu.CompilerParams(
            dimension_semantics=("parallel",)),
        interpret=interpret,
    )(annotations, anc_t, reg_t)

    ann_t = jnp.swapaxes(annotations, 1, 2)
    acc_b = pl.pallas_call(
        _cls_body,
        grid=(B, _TC_NB),
        in_specs=[
            pl.BlockSpec((1, 5, _G), lambda j, i: (j, 0, 0)),
            pl.BlockSpec((1, _BA, C), lambda j, i: (j, i, 0)),
            pl.BlockSpec((1, _BA, 4), lambda j, i: (0, i, 0)),
        ],
        out_specs=pl.BlockSpec((1, 8, 128), lambda j, i: (j, 0, 0)),
        out_shape=jax.ShapeDtypeStruct((B, 8, 128), jnp.float32),
        compiler_params=pltpu.CompilerParams(
            dimension_semantics=("parallel", "arbitrary")),
        interpret=interpret,
    )(ann_t, classifications, anchors)

    cls_sc = jnp.sum(jnp.reshape(sc_acc, (B, 16 * 16)), axis=1)
    cls_sum = acc_b[:, 0, 0] + cls_sc
    reg_sum = acc_a[:, 0, 1]
    npos = acc_a[:, 0, 2]
    cls_loss = cls_sum / jnp.maximum(npos, 1.0)
    reg_loss = reg_sum / jnp.maximum(npos * 4.0, 1.0)
    return (jnp.mean(cls_loss, keepdims=True),
            jnp.mean(reg_loss, keepdims=True))


# speedup vs baseline: 1.1815x; 1.1815x over previous
import functools

import jax
import jax.numpy as jnp
from jax import lax
from jax.experimental import pallas as pl
from jax.experimental.pallas import tpu as pltpu
from jax.experimental.pallas import tpu_sc as plsc

_ALPHA = 0.25
_A_TOTAL = 49104
_A_PAD = 49152
_C = 80
_G = 20
_BA = 4096
_TC_LIM = 28624
_TC_NB = 7
_SC_N = _A_TOTAL - _TC_LIM
_SC_PER_W = _SC_N // 16
_SC_NCH = 4
_SC_CHA = _SC_PER_W // _SC_NCH
_SC_CH = _SC_CHA * _C

_L2C = (-2.786812953867443, 5.046876044975941, -3.49249427987935,
        1.5939013634991297, -0.4048671744191854, 0.043428907822139526)
_LN2 = 0.6931471805599453


def _assign_body(ann_ref, anc_ref, reg_ref, acc_ref):
    j = pl.program_id(0)
    sa = anc_ref.shape[1]

    ax1 = anc_ref[0]
    ay1 = anc_ref[1]
    ax2 = anc_ref[2]
    ay2 = anc_ref[3]
    area_a = (ax2 - ax1) * (ay2 - ay1)

    best = jnp.full((sa, 128), -2.0, jnp.float32)
    gx1 = jnp.zeros((sa, 128), jnp.float32)
    gy1 = gx1
    gx2 = gx1
    gy2 = gx1
    for g in range(_G):
        bx1 = ann_ref[j, g, 0]
        by1 = ann_ref[j, g, 1]
        bx2 = ann_ref[j, g, 2]
        by2 = ann_ref[j, g, 3]
        bcl = ann_ref[j, g, 4]
        iw = jnp.clip(jnp.minimum(ax2, bx2) - jnp.maximum(ax1, bx1), 0.0, None)
        ih = jnp.clip(jnp.minimum(ay2, by2) - jnp.maximum(ay1, by1), 0.0, None)
        inter = iw * ih
        ua = jnp.clip(area_a + (bx2 - bx1) * (by2 - by1) - inter, 1e-8, None)
        iou = jnp.where(bcl != -1.0, inter / ua, -1.0)
        upd = iou > best
        best = jnp.where(upd, iou, best)
        gx1 = jnp.where(upd, bx1, gx1)
        gy1 = jnp.where(upd, by1, gy1)
        gx2 = jnp.where(upd, bx2, gx2)
        gy2 = jnp.where(upd, by2, gy2)

    lane = jax.lax.broadcasted_iota(jnp.int32, (sa, 128), 1)
    sub = jax.lax.broadcasted_iota(jnp.int32, (sa, 128), 0)
    rv = sub * 128 + lane < _A_TOTAL
    posv = jnp.logical_and(best >= 0.5, rv)

    aw = ax2 - ax1
    ah = ay2 - ay1
    gwr = gx2 - gx1
    ghr = gy2 - gy1
    tdx = (gx1 + 0.5 * gwr - ax1 - 0.5 * aw) / aw * 10.0
    tdy = (gy1 + 0.5 * ghr - ay1 - 0.5 * ah) / ah * 10.0
    tdw = jnp.log(jnp.clip(gwr, 1.0, None) / aw) * 5.0
    tdh = jnp.log(jnp.clip(ghr, 1.0, None) / ah) * 5.0

    def _sl1(d):
        ad = jnp.abs(d)
        return jnp.where(ad <= 1.0 / 9.0, 4.5 * ad * ad, ad - 0.5 / 9.0)

    rl = (_sl1(tdx - reg_ref[0, 0]) + _sl1(tdy - reg_ref[0, 1])
          + _sl1(tdw - reg_ref[0, 2]) + _sl1(tdh - reg_ref[0, 3]))
    reg_part = jnp.sum(jnp.where(posv, rl, 0.0))
    pos_part = jnp.sum(jnp.where(posv, 1.0, 0.0))
    c128 = jax.lax.broadcasted_iota(jnp.int32, (8, 128), 1)
    acc_ref[0] = (jnp.where(c128 == 1, reg_part, 0.0)
                  + jnp.where(c128 == 2, pos_part, 0.0))


def _cls_body(ann_ref, cls_ref, anc_ref, out_ref):
    i = pl.program_id(1)

    @pl.when(i == 0)
    def _init():
        out_ref[...] = jnp.zeros_like(out_ref)

    bx1 = ann_ref[0, 0:1, :]
    by1 = ann_ref[0, 1:2, :]
    bx2 = ann_ref[0, 2:3, :]
    by2 = ann_ref[0, 3:4, :]
    bcl = ann_ref[0, 4:5, :]

    ax1 = anc_ref[0, :, 0:1]
    ay1 = anc_ref[0, :, 1:2]
    ax2 = anc_ref[0, :, 2:3]
    ay2 = anc_ref[0, :, 3:4]

    iw = jnp.clip(jnp.minimum(ax2, bx2) - jnp.maximum(ax1, bx1), 0.0, None)
    ih = jnp.clip(jnp.minimum(ay2, by2) - jnp.maximum(ay1, by1), 0.0, None)
    inter = iw * ih
    area_b = (bx2 - bx1) * (by2 - by1)
    area_a = (ax2 - ax1) * (ay2 - ay1)
    ua = jnp.clip(area_a + area_b - inter, 1e-8, None)
    iou = jnp.where(bcl != -1.0, inter / ua, -1.0)

    iou_max = jnp.max(iou, axis=1, keepdims=True)
    g_iota = jax.lax.broadcasted_iota(jnp.int32, iou.shape, 1)
    first_arg = jnp.min(jnp.where(iou >= iou_max, g_iota, 2**30), axis=1,
                        keepdims=True)
    eq = (g_iota == first_arg).astype(jnp.float32)
    gcl = jnp.sum(eq * bcl, axis=1, keepdims=True)

    row = jax.lax.broadcasted_iota(jnp.int32, iou_max.shape, 0) + i * _BA
    bnd = row < _TC_LIM
    pc = jnp.logical_and(iou_max >= 0.5, bnd)
    kc = jnp.logical_and(jnp.logical_or(iou_max < 0.4, iou_max >= 0.5), bnd)
    ic = gcl.astype(jnp.int32)

    c = jnp.clip(cls_ref[0], 1e-4, 1.0 - 1e-4)
    neg = (1.0 - _ALPHA) * c * c * (-jnp.log(1.0 - c))
    c_iota = jax.lax.broadcasted_iota(jnp.int32, c.shape, 1)
    onehot = c_iota == ic
    s1 = jnp.sum(jnp.where(kc, neg, 0.0))
    s_negt = jnp.sum(jnp.where(jnp.logical_and(onehot, pc), neg, 0.0))
    ct = jnp.sum(jnp.where(onehot, c, 0.0), axis=1, keepdims=True)
    ct = jnp.clip(ct, 1e-4, 1.0 - 1e-4)
    post = _ALPHA * (1.0 - ct) * (1.0 - ct) * (-jnp.log(ct))
    s_post = jnp.sum(jnp.where(pc, post, 0.0))
    cls_part = s1 - s_negt + s_post

    c128 = jax.lax.broadcasted_iota(jnp.int32, (8, 128), 1)
    out_ref[0] += jnp.where(c128 == 0, cls_part, 0.0)


def _fast_log(y):
    bits = plsc.bitcast(y, jnp.int32)
    e = (jnp.right_shift(bits, 23) - 127).astype(jnp.float32)
    m = plsc.bitcast(
        jnp.bitwise_or(jnp.bitwise_and(bits, 0x007FFFFF), 0x3F800000),
        jnp.float32)
    p = jnp.float32(_L2C[5])
    for k in (4, 3, 2, 1, 0):
        p = p * m + jnp.float32(_L2C[k])
    return (e + p) * jnp.float32(_LN2)


def _sc_cls_body(cls_hbm, anc_hbm, ann_hbm, out_hbm,
                 buf0, buf1, ancbuf, abuf, accv, sem0, sem1):
    w = lax.axis_index("s") * 2 + lax.axis_index("c")
    b = w // 16
    wis = w % 16
    abase = _TC_LIM + wis * _SC_PER_W
    ebase = b * (_A_TOTAL * _C) + abase * _C

    pltpu.sync_copy(ann_hbm, abuf)
    pltpu.sync_copy(anc_hbm.at[pl.ds(abase * 4, _SC_PER_W * 4)], ancbuf)

    bufs = (buf0, buf1)
    sems = (sem0, sem1)
    pltpu.make_async_copy(cls_hbm.at[pl.ds(ebase, _SC_CH)], buf0, sem0).start()

    iv = lax.iota(jnp.int32, 16) * _C
    iv4 = lax.iota(jnp.int32, 16) * 4
    zero16 = jnp.zeros((16,), jnp.int32)
    carry0 = (jnp.zeros((16,), jnp.float32),) * 3

    def _mk_group(cur, coff):
        def _group(g, carry):
            sneg, snt, spost = carry
            la4 = (coff + g * 16) * 4
            ax1 = plsc.load_gather(ancbuf, [iv4 + la4])
            ay1 = plsc.load_gather(ancbuf, [iv4 + (la4 + 1)])
            ax2 = plsc.load_gather(ancbuf, [iv4 + (la4 + 2)])
            ay2 = plsc.load_gather(ancbuf, [iv4 + (la4 + 3)])
            area_a = (ax2 - ax1) * (ay2 - ay1)
            best = jnp.full((16,), -2.0, jnp.float32)
            gcl = jnp.zeros((16,), jnp.float32)
            for gt in range(_G):
                bann = b * (_G * 5) + gt * 5
                bx1 = plsc.load_gather(abuf, [zero16 + bann])
                by1 = plsc.load_gather(abuf, [zero16 + (bann + 1)])
                bx2 = plsc.load_gather(abuf, [zero16 + (bann + 2)])
                by2 = plsc.load_gather(abuf, [zero16 + (bann + 3)])
                bcl = plsc.load_gather(abuf, [zero16 + (bann + 4)])
                iw = jnp.maximum(jnp.minimum(ax2, bx2) - jnp.maximum(ax1, bx1),
                                 0.0)
                ih = jnp.maximum(jnp.minimum(ay2, by2) - jnp.maximum(ay1, by1),
                                 0.0)
                inter = iw * ih
                ua = jnp.maximum(
                    area_a + (bx2 - bx1) * (by2 - by1) - inter, 1e-8)
                iou = jnp.where(bcl != -1.0, inter / ua, -1.0)
                upd = iou > best
                best = jnp.where(upd, iou, best)
                gcl = jnp.where(upd, bcl, gcl)
            posb = best >= 0.5
            keepb = jnp.logical_or(best < 0.4, posb)
            eidx = iv + g * (16 * _C)
            for k in range(_C):
                ck = plsc.load_gather(cur, [eidx + k])
                cc = jnp.minimum(jnp.maximum(ck, 1e-4), 1.0 - 1e-4)
                negk = (_ALPHA - 1.0) * cc * cc * _fast_log(1.0 - cc)
                sneg = sneg + jnp.where(keepb, negk, 0.0)
            ct = plsc.load_gather(
                cur, [eidx + jnp.maximum(gcl, 0.0).astype(jnp.int32)])
            ct = jnp.minimum(jnp.maximum(ct, 1e-4), 1.0 - 1e-4)
            negt = (_ALPHA - 1.0) * ct * ct * _fast_log(1.0 - ct)
            snt = snt + jnp.where(posb, negt, 0.0)
            post = -_ALPHA * (1.0 - ct) * (1.0 - ct) * _fast_log(ct)
            spost = spost + jnp.where(posb, post, 0.0)
            return (sneg, snt, spost)
        return _group

    carry = carry0
    for c in range(_SC_NCH):
        if c + 1 < _SC_NCH:
            pltpu.make_async_copy(
                cls_hbm.at[pl.ds(ebase + (c + 1) * _SC_CH, _SC_CH)],
                bufs[(c + 1) % 2], sems[(c + 1) % 2]).start()
        pltpu.make_async_copy(cls_hbm.at[pl.ds(ebase + c * _SC_CH, _SC_CH)],
                              bufs[c % 2], sems[c % 2]).wait()
        carry = lax.fori_loop(0, _SC_CHA // 16,
                              _mk_group(bufs[c % 2], c * _SC_CHA), carry)

    sneg, snt, spost = carry
    accv[...] = sneg - snt + spost
    pltpu.sync_copy(accv, out_hbm.at[w])


@functools.partial(jax.jit, static_argnames=("interpret",))
def kernel(classifications, regressions, anchors, annotations, image,
           interpret=False):
    del image
    B, A, C = classifications.shape
    sa = _A_PAD // 128

    mesh = plsc.VectorSubcoreMesh(core_axis_name="c", subcore_axis_name="s")
    sc_cls = functools.partial(
        pl.kernel,
        out_type=jax.ShapeDtypeStruct((32, 16), jnp.float32),
        mesh=mesh,
        scratch_types=[
            pltpu.VMEM((_SC_CH,), jnp.float32),
            pltpu.VMEM((_SC_CH,), jnp.float32),
            pltpu.VMEM((_SC_PER_W * 4,), jnp.float32),
            pltpu.VMEM((B * _G * 5,), jnp.float32),
            pltpu.VMEM((16,), jnp.float32),
            pltpu.SemaphoreType.DMA,
            pltpu.SemaphoreType.DMA,
        ],
        compiler_params=pltpu.CompilerParams(needs_layout_passes=False),
        interpret=interpret,
    )(_sc_cls_body)
    sc_acc = sc_cls(jnp.reshape(classifications, (-1,)),
                    jnp.reshape(anchors[0], (-1,)),
                    jnp.reshape(annotations, (-1,)))

    anc_t = jnp.swapaxes(anchors[0], 0, 1)
    anc_t = jnp.reshape(jnp.pad(anc_t, ((0, 0), (0, _A_PAD - A))),
                        (4, sa, 128))
    reg_t = jnp.swapaxes(regressions, 1, 2)
    reg_t = jnp.reshape(jnp.pad(reg_t, ((0, 0), (0, 0), (0, _A_PAD - A))),
                        (B, 4, sa, 128))
    acc_a = pl.pallas_call(
        _assign_body,
        grid=(B,),
        in_specs=[
            pl.BlockSpec(memory_space=pltpu.SMEM),
            pl.BlockSpec((4, sa, 128), lambda j: (0, 0, 0)),
            pl.BlockSpec((1, 4, sa, 128), lambda j: (j, 0, 0, 0)),
        ],
        out_specs=pl.BlockSpec((1, 8, 128), lambda j: (j, 0, 0)),
        out_shape=jax.ShapeDtypeStruct((B, 8, 128), jnp.float32),
        compiler_params=pltpu.CompilerParams(
            dimension_semantics=("parallel",)),
        interpret=interpret,
    )(annotations, anc_t, reg_t)

    ann_t = jnp.swapaxes(annotations, 1, 2)
    acc_b = pl.pallas_call(
        _cls_body,
        grid=(B, _TC_NB),
        in_specs=[
            pl.BlockSpec((1, 5, _G), lambda j, i: (j, 0, 0)),
            pl.BlockSpec((1, _BA, C), lambda j, i: (j, i, 0)),
            pl.BlockSpec((1, _BA, 4), lambda j, i: (0, i, 0)),
        ],
        out_specs=pl.BlockSpec((1, 8, 128), lambda j, i: (j, 0, 0)),
        out_shape=jax.ShapeDtypeStruct((B, 8, 128), jnp.float32),
        compiler_params=pltpu.CompilerParams(
            dimension_semantics=("parallel", "arbitrary")),
        interpret=interpret,
    )(ann_t, classifications, anchors)

    cls_sc = jnp.sum(jnp.reshape(sc_acc, (B, 16 * 16)), axis=1)
    cls_sum = acc_b[:, 0, 0] + cls_sc
    reg_sum = acc_a[:, 0, 1]
    npos = acc_a[:, 0, 2]
    cls_loss = cls_sum / jnp.maximum(npos, 1.0)
    reg_loss = reg_sum / jnp.maximum(npos * 4.0, 1.0)
    return (jnp.mean(cls_loss, keepdims=True),
            jnp.mean(reg_loss, keepdims=True))
